# Initial kernel scaffold; baseline (speedup 1.0000x reference)
import jax
import jax.numpy as jnp
from jax.experimental import pallas as pl


def kernel(x, adj, W1, b1, g1, beta1, W2, b2, g2, beta2, W3, b3, g3, beta3, Wl, bl):
    raise NotImplementedError("write your pallas kernel here")


# speedup vs baseline: 1.5115x; 1.5115x over previous
import functools

import jax
import jax.numpy as jnp
from jax.experimental import pallas as pl

_EPS = 1e-5


def _stats_to_affine(s_ref, g_ref, beta_ref, m):
    mean = s_ref[0:1, :] / m
    var = s_ref[1:2, :] / m - mean * mean
    inv = jax.lax.rsqrt(var + _EPS)
    a = g_ref[...] * inv
    c = beta_ref[...] - mean * a
    return a, c


def _accum_stats(y, s_ref):
    part = jnp.concatenate(
        [jnp.sum(y, axis=0, keepdims=True), jnp.sum(y * y, axis=0, keepdims=True)],
        axis=0,
    )

    @pl.when(pl.program_id(0) == 0)
    def _():
        s_ref[...] = part

    @pl.when(pl.program_id(0) != 0)
    def _():
        s_ref[...] += part


def _layer1_body(x_ref, W_ref, b_ref, y_ref, s_ref):
    y = jax.nn.relu(
        jnp.dot(x_ref[...], W_ref[...], preferred_element_type=jnp.float32)
        + b_ref[...]
    )
    y_ref[...] = y
    _accum_stats(y, s_ref)


def _mid_body(y_in_ref, s_in_ref, g_ref, beta_ref, W_ref, b_ref, y_ref, s_ref, *, m):
    a, c = _stats_to_affine(s_in_ref, g_ref, beta_ref, m)
    xn = y_in_ref[...] * a + c
    y = jax.nn.relu(
        jnp.dot(xn, W_ref[...], preferred_element_type=jnp.float32) + b_ref[...]
    )
    y_ref[...] = y
    _accum_stats(y, s_ref)


def _head_body(
    y1_ref, y2_ref, y3_ref,
    s1_ref, g1_ref, beta1_ref,
    s2_ref, g2_ref, beta2_ref,
    s3_ref, g3_ref, beta3_ref,
    Wl_ref, bl_ref, out_ref, *, m, h,
):
    a1, c1 = _stats_to_affine(s1_ref, g1_ref, beta1_ref, m)
    a2, c2 = _stats_to_affine(s2_ref, g2_ref, beta2_ref, m)
    a3, c3 = _stats_to_affine(s3_ref, g3_ref, beta3_ref, m)
    x1 = y1_ref[...] * a1 + c1
    x2 = y2_ref[...] * a2 + c2
    x3 = y3_ref[...] * a3 + c3
    acc = jnp.dot(x1, Wl_ref[0:h, :], preferred_element_type=jnp.float32)
    acc += jnp.dot(x2, Wl_ref[h : 2 * h, :], preferred_element_type=jnp.float32)
    acc += jnp.dot(x3, Wl_ref[2 * h :, :], preferred_element_type=jnp.float32)
    out_ref[...] = jax.nn.relu(acc + bl_ref[...])


def _full(shape):
    return pl.BlockSpec(shape, lambda i: (0, 0))


def _rows(r, c):
    return pl.BlockSpec((r, c), lambda i: (i, 0))


def kernel(x, adj, W1, b1, g1, beta1, W2, b2, g2, beta2, W3, b3, g3, beta3, Wl, bl):
    del adj
    B, N, Cin = x.shape
    H = W1.shape[1]
    Cout = W3.shape[1]
    M = B * N
    R = 2048
    nb = M // R
    grid = (nb,)
    mf = float(M)

    xf = x.reshape(M, Cin)
    row = lambda v: v.reshape(1, -1)

    y1, s1 = pl.pallas_call(
        _layer1_body,
        grid=grid,
        in_specs=[_rows(R, Cin), _full((Cin, H)), _full((1, H))],
        out_specs=[_rows(R, H), _full((2, H))],
        out_shape=[
            jax.ShapeDtypeStruct((M, H), jnp.float32),
            jax.ShapeDtypeStruct((2, H), jnp.float32),
        ],
    )(xf, W1, row(b1))

    mid = functools.partial(_mid_body, m=mf)
    y2, s2 = pl.pallas_call(
        mid,
        grid=grid,
        in_specs=[_rows(R, H), _full((2, H)), _full((1, H)), _full((1, H)),
                  _full((H, H)), _full((1, H))],
        out_specs=[_rows(R, H), _full((2, H))],
        out_shape=[
            jax.ShapeDtypeStruct((M, H), jnp.float32),
            jax.ShapeDtypeStruct((2, H), jnp.float32),
        ],
    )(y1, s1, row(g1), row(beta1), W2, row(b2))

    y3, s3 = pl.pallas_call(
        mid,
        grid=grid,
        in_specs=[_rows(R, H), _full((2, H)), _full((1, H)), _full((1, H)),
                  _full((H, Cout)), _full((1, Cout))],
        out_specs=[_rows(R, Cout), _full((2, Cout))],
        out_shape=[
            jax.ShapeDtypeStruct((M, Cout), jnp.float32),
            jax.ShapeDtypeStruct((2, Cout), jnp.float32),
        ],
    )(y2, s2, row(g2), row(beta2), W3, row(b3))

    out = pl.pallas_call(
        functools.partial(_head_body, m=mf, h=H),
        grid=grid,
        in_specs=[
            _rows(R, H), _rows(R, H), _rows(R, Cout),
            _full((2, H)), _full((1, H)), _full((1, H)),
            _full((2, H)), _full((1, H)), _full((1, H)),
            _full((2, Cout)), _full((1, Cout)), _full((1, Cout)),
            _full((2 * H + Cout, Cout)), _full((1, Cout)),
        ],
        out_specs=_rows(R, Cout),
        out_shape=jax.ShapeDtypeStruct((M, Cout), jnp.float32),
    )(
        y1, y2, y3,
        s1, row(g1), row(beta1),
        s2, row(g2), row(beta2),
        s3, row(g3), row(beta3),
        Wl, row(bl),
    )

    return out.reshape(B, N, Cout)


# speedup vs baseline: 1.8867x; 1.2482x over previous
import functools

import jax
import jax.numpy as jnp
from jax.experimental import pallas as pl

_EPS = 1e-5


def _stats_to_affine(s_ref, g_ref, beta_ref, m):
    mean = s_ref[0:1, :] / m
    var = s_ref[1:2, :] / m - mean * mean
    inv = jax.lax.rsqrt(var + _EPS)
    a = g_ref[...] * inv
    c = beta_ref[...] - mean * a
    return a, c


def _accum_stats(y, s_ref):
    part = jnp.concatenate(
        [jnp.sum(y, axis=0, keepdims=True), jnp.sum(y * y, axis=0, keepdims=True)],
        axis=0,
    )

    @pl.when(pl.program_id(0) == 0)
    def _():
        s_ref[...] = part

    @pl.when(pl.program_id(0) != 0)
    def _():
        s_ref[...] += part


def _layer1_body(x_ref, W_ref, b_ref, y_ref, s_ref):
    y = jax.nn.relu(
        jnp.dot(x_ref[...], W_ref[...], preferred_element_type=jnp.float32)
        + b_ref[...]
    )
    y_ref[...] = y.astype(jnp.bfloat16)
    _accum_stats(y, s_ref)


def _mid_body(y_in_ref, s_in_ref, g_ref, beta_ref, W_ref, b_ref, y_ref, s_ref, *, m):
    a, c = _stats_to_affine(s_in_ref, g_ref, beta_ref, m)
    xn = y_in_ref[...].astype(jnp.float32) * a + c
    y = jax.nn.relu(
        jnp.dot(xn, W_ref[...], preferred_element_type=jnp.float32) + b_ref[...]
    )
    y_ref[...] = y.astype(jnp.bfloat16)
    _accum_stats(y, s_ref)


def _head_body(
    y1_ref, y2_ref, y3_ref,
    s1_ref, g1_ref, beta1_ref,
    s2_ref, g2_ref, beta2_ref,
    s3_ref, g3_ref, beta3_ref,
    Wl_ref, bl_ref, out_ref, *, m, h,
):
    a1, c1 = _stats_to_affine(s1_ref, g1_ref, beta1_ref, m)
    a2, c2 = _stats_to_affine(s2_ref, g2_ref, beta2_ref, m)
    a3, c3 = _stats_to_affine(s3_ref, g3_ref, beta3_ref, m)
    x1 = y1_ref[...].astype(jnp.float32) * a1 + c1
    x2 = y2_ref[...].astype(jnp.float32) * a2 + c2
    x3 = y3_ref[...].astype(jnp.float32) * a3 + c3
    acc = jnp.dot(x1, Wl_ref[0:h, :], preferred_element_type=jnp.float32)
    acc += jnp.dot(x2, Wl_ref[h : 2 * h, :], preferred_element_type=jnp.float32)
    acc += jnp.dot(x3, Wl_ref[2 * h :, :], preferred_element_type=jnp.float32)
    out_ref[...] = jax.nn.relu(acc + bl_ref[...])


def _full(shape):
    return pl.BlockSpec(shape, lambda i: (0, 0))


def _rows(r, c):
    return pl.BlockSpec((r, c), lambda i: (i, 0))


def kernel(x, adj, W1, b1, g1, beta1, W2, b2, g2, beta2, W3, b3, g3, beta3, Wl, bl):
    del adj
    B, N, Cin = x.shape
    H = W1.shape[1]
    Cout = W3.shape[1]
    M = B * N
    R = 2048
    nb = M // R
    grid = (nb,)
    mf = float(M)

    xf = x.reshape(M, Cin)
    row = lambda v: v.reshape(1, -1)

    y1, s1 = pl.pallas_call(
        _layer1_body,
        grid=grid,
        in_specs=[_rows(R, Cin), _full((Cin, H)), _full((1, H))],
        out_specs=[_rows(R, H), _full((2, H))],
        out_shape=[
            jax.ShapeDtypeStruct((M, H), jnp.bfloat16),
            jax.ShapeDtypeStruct((2, H), jnp.float32),
        ],
    )(xf, W1, row(b1))

    mid = functools.partial(_mid_body, m=mf)
    y2, s2 = pl.pallas_call(
        mid,
        grid=grid,
        in_specs=[_rows(R, H), _full((2, H)), _full((1, H)), _full((1, H)),
                  _full((H, H)), _full((1, H))],
        out_specs=[_rows(R, H), _full((2, H))],
        out_shape=[
            jax.ShapeDtypeStruct((M, H), jnp.bfloat16),
            jax.ShapeDtypeStruct((2, H), jnp.float32),
        ],
    )(y1, s1, row(g1), row(beta1), W2, row(b2))

    y3, s3 = pl.pallas_call(
        mid,
        grid=grid,
        in_specs=[_rows(R, H), _full((2, H)), _full((1, H)), _full((1, H)),
                  _full((H, Cout)), _full((1, Cout))],
        out_specs=[_rows(R, Cout), _full((2, Cout))],
        out_shape=[
            jax.ShapeDtypeStruct((M, Cout), jnp.bfloat16),
            jax.ShapeDtypeStruct((2, Cout), jnp.float32),
        ],
    )(y2, s2, row(g2), row(beta2), W3, row(b3))

    out = pl.pallas_call(
        functools.partial(_head_body, m=mf, h=H),
        grid=grid,
        in_specs=[
            _rows(R, H), _rows(R, H), _rows(R, Cout),
            _full((2, H)), _full((1, H)), _full((1, H)),
            _full((2, H)), _full((1, H)), _full((1, H)),
            _full((2, Cout)), _full((1, Cout)), _full((1, Cout)),
            _full((2 * H + Cout, Cout)), _full((1, Cout)),
        ],
        out_specs=_rows(R, Cout),
        out_shape=jax.ShapeDtypeStruct((M, Cout), jnp.float32),
    )(
        y1, y2, y3,
        s1, row(g1), row(beta1),
        s2, row(g2), row(beta2),
        s3, row(g3), row(beta3),
        Wl, row(bl),
    )

    return out.reshape(B, N, Cout)
